# baseline (speedup 1.0000x reference)
import functools

import jax
import jax.numpy as jnp
from jax import lax
from jax.experimental import pallas as pl
from jax.experimental.pallas import tpu as pltpu
from jax.experimental.pallas import tpu_sc as plsc

NTOKEN = 1000000
NINP = 64
NC = 2
NS = 16
NW = NC * NS
CH = 128
NBUF = 4


def _embed_body(table_hbm, idx_hbm, out_hbm, idx_v, *rest):
    nch = idx_hbm.shape[1]
    ngroups = nch // NBUF
    rows = rest[:NBUF]
    sems = rest[NBUF:]

    wid = lax.axis_index("s") * NC + lax.axis_index("c")
    pltpu.sync_copy(idx_hbm.at[wid], idx_v)

    def fire(j, b):
        pltpu.async_copy(table_hbm.at[idx_v.at[j]], rows[b], sems[b])

    def drain(j, b):
        pltpu.make_async_copy(table_hbm.at[idx_v.at[j]], rows[b], sems[b]).wait()
        pltpu.sync_copy(rows[b], out_hbm.at[wid, j])

    for b in range(NBUF):
        fire(b, b)

    def group(g, carry):
        for b in range(NBUF):
            j = g * NBUF + b
            drain(j, b)
            fire(j + NBUF, b)
        return carry

    lax.fori_loop(0, ngroups - 1, group, None)

    for b in range(NBUF):
        drain((ngroups - 1) * NBUF + b, b)


def _make_gather(nch):
    mesh = plsc.VectorSubcoreMesh(
        core_axis_name="c", subcore_axis_name="s",
        num_cores=NC, num_subcores=NS)
    scratch = [pltpu.VMEM((nch, CH), jnp.int32)]
    scratch += [pltpu.VMEM((CH, NINP), jnp.float32) for _ in range(NBUF)]
    scratch += [pltpu.SemaphoreType.DMA for _ in range(NBUF)]
    return pl.kernel(
        _embed_body,
        out_type=jax.ShapeDtypeStruct((NW, nch, CH, NINP), jnp.float32),
        mesh=mesh,
        scratch_types=scratch,
        compiler_params=pltpu.CompilerParams(use_tc_tiling_on_sc=False),
    )


@jax.jit
def kernel(input, weight):
    b, l = input.shape
    nch = (b * l) // (NW * CH)
    idx = input.reshape(NW, nch, CH).astype(jnp.int32)
    out = _make_gather(nch)(weight, idx)
    return out.reshape(b, l, NINP)


# speedup vs baseline: 1.0001x; 1.0001x over previous
import functools

import jax
import jax.numpy as jnp
from jax import lax
from jax.experimental import pallas as pl
from jax.experimental.pallas import tpu as pltpu
from jax.experimental.pallas import tpu_sc as plsc

NTOKEN = 1000000
NINP = 64
NC = 2
NS = 16
NW = NC * NS
CH = 128
NBUF = 4


def _embed_body(table_hbm, idx_hbm, out_hbm, idx_v, *rest):
    nch = idx_hbm.shape[1]
    ngroups = nch // NBUF
    rows = rest[:NBUF]
    sems = rest[NBUF:]

    wid = lax.axis_index("s") * NC + lax.axis_index("c")
    pltpu.sync_copy(idx_hbm.at[wid], idx_v)

    def fire(j, b):
        pltpu.async_copy(table_hbm.at[idx_v.at[j]], rows[b], sems[b])

    def drain(j, b):
        pltpu.make_async_copy(table_hbm.at[idx_v.at[j]], rows[b], sems[b]).wait()
        pltpu.sync_copy(rows[b], out_hbm.at[pl.ds((wid * nch + j) * CH, CH)])

    for b in range(NBUF):
        fire(b, b)

    def group(g, carry):
        for b in range(NBUF):
            j = g * NBUF + b
            drain(j, b)
            fire(j + NBUF, b)
        return carry

    lax.fori_loop(0, ngroups - 1, group, None)

    for b in range(NBUF):
        drain((ngroups - 1) * NBUF + b, b)


def _make_gather(nch):
    mesh = plsc.VectorSubcoreMesh(
        core_axis_name="c", subcore_axis_name="s",
        num_cores=NC, num_subcores=NS)
    scratch = [pltpu.VMEM((nch, CH), jnp.int32)]
    scratch += [pltpu.VMEM((CH, NINP), jnp.float32) for _ in range(NBUF)]
    scratch += [pltpu.SemaphoreType.DMA for _ in range(NBUF)]
    return pl.kernel(
        _embed_body,
        out_type=jax.ShapeDtypeStruct((NW * nch * CH, NINP), jnp.float32),
        mesh=mesh,
        scratch_types=scratch,
        compiler_params=pltpu.CompilerParams(use_tc_tiling_on_sc=False),
    )


@jax.jit
def kernel(input, weight):
    b, l = input.shape
    nch = (b * l) // (NW * CH)
    idx = input.reshape(NW, nch, CH).astype(jnp.int32)
    out = _make_gather(nch)(weight, idx)
    return out.reshape(b, l, NINP)


# speedup vs baseline: 1.0523x; 1.0522x over previous
import functools

import jax
import jax.numpy as jnp
from jax import lax
from jax.experimental import pallas as pl
from jax.experimental.pallas import tpu as pltpu
from jax.experimental.pallas import tpu_sc as plsc

NTOKEN = 1000000
NINP = 64
NC = 2
NS = 16
NW = NC * NS
CH = 128
NBUF = 2
NDBUF = 2


def _embed_body(w2_hbm, p_hbm, h_hbm, out_hbm, p_v, h_v, *rest):
    n = p_hbm.shape[0] // NW
    nch = n // CH
    gbufs = rest[:NBUF]
    dbufs = rest[NBUF:NBUF + NDBUF]
    gsems = rest[NBUF + NDBUF:NBUF + NDBUF + NBUF]
    wsems = rest[NBUF + NDBUF + NBUF:]

    wid = lax.axis_index("s") * NC + lax.axis_index("c")
    base = wid * n
    pltpu.sync_copy(p_hbm.at[pl.ds(base, n)], p_v)
    pltpu.sync_copy(h_hbm.at[pl.ds(base, n)], h_v)

    def fire(j, b):
        pltpu.async_copy(
            w2_hbm.at[p_v.at[pl.ds(j * CH, CH)]], gbufs[b], gsems[b])

    def wait_gather(j, b):
        pltpu.make_async_copy(
            w2_hbm.at[p_v.at[pl.ds(j * CH, CH)]], gbufs[b], gsems[b]).wait()

    def select(j, b, d):
        def grp(g, carry):
            hvec = h_v[pl.ds(j * CH + 16 * g, 16)]
            for kk in range(16):
                h = hvec[kk]
                hr = h >> 1
                hc = (h & 1) * NINP
                k = 16 * g + kk
                for i in range(NINP // 16):
                    dbufs[d][k, pl.ds(16 * i, 16)] = (
                        gbufs[b][hr, pl.ds(hc + 16 * i, 16)])
            return carry
        lax.fori_loop(0, CH // 16, grp, None)

    def put(j, d):
        pltpu.async_copy(
            dbufs[d], out_hbm.at[pl.ds((wid * nch + j) * CH, CH)], wsems[d])

    def wait_put(j, d):
        pltpu.make_async_copy(
            dbufs[d], out_hbm.at[pl.ds((wid * nch + j) * CH, CH)],
            wsems[d]).wait()

    for b in range(NBUF):
        fire(b, b)

    def group(g, carry):
        for b in range(NBUF):
            j = g * NBUF + b
            wait_gather(j, b)
            pl.when(j >= NDBUF)(lambda: wait_put(j - NDBUF, b))
            select(j, b, b)
            put(j, b)
            fire(j + NBUF, b)
        return carry

    lax.fori_loop(0, nch // NBUF - 1, group, None)

    for b in range(NBUF):
        j = (nch // NBUF - 1) * NBUF + b
        wait_gather(j, b)
        wait_put(j - NDBUF, b)
        select(j, b, b)
        put(j, b)
    for b in range(NBUF):
        wait_put((nch // NBUF - 1) * NBUF + b, b)


def _make_gather(n_total):
    nch = n_total // (NW * CH)
    mesh = plsc.VectorSubcoreMesh(
        core_axis_name="c", subcore_axis_name="s",
        num_cores=NC, num_subcores=NS)
    per_w = n_total // NW
    scratch = [
        pltpu.VMEM((per_w,), jnp.int32),
        pltpu.VMEM((per_w,), jnp.int32),
    ]
    scratch += [pltpu.VMEM((CH, 128), jnp.float32) for _ in range(NBUF)]
    scratch += [pltpu.VMEM((CH, NINP), jnp.float32) for _ in range(NDBUF)]
    scratch += [pltpu.SemaphoreType.DMA for _ in range(NBUF)]
    scratch += [pltpu.SemaphoreType.DMA for _ in range(NDBUF)]
    return pl.kernel(
        _embed_body,
        out_type=jax.ShapeDtypeStruct((NW * nch * CH, NINP), jnp.float32),
        mesh=mesh,
        scratch_types=scratch,
        compiler_params=pltpu.CompilerParams(use_tc_tiling_on_sc=True),
    )


@jax.jit
def kernel(input, weight):
    b, l = input.shape
    n_total = b * l
    flat = input.reshape(n_total).astype(jnp.int32)
    p = flat >> 1
    h = 2 * (jnp.arange(n_total, dtype=jnp.int32) % CH) + (flat & 1)
    w2 = weight.reshape(NTOKEN // 2, 128)
    out = _make_gather(n_total)(w2, p, h)
    return out.reshape(b, l, NINP)
